# baseline (speedup 1.0000x reference)
import jax
import jax.numpy as jnp
from jax import lax
from jax.experimental import pallas as pl
from jax.experimental.pallas import tpu as pltpu
from jax.experimental.pallas import tpu_sc as plsc

N_PREFIX = 10000
N_ITEM = 10000
D = 128
NC, NS = 2, 16
NW = NC * NS
CHUNK = 128
ROWS_PER_TILE = 632
NPAD = NS * ROWS_PER_TILE
DUMMY_ROW = NPAD - 1


def _prep_body(h_ref, w_ref, a_ref, hw_ref, item_ref):
    h = h_ref[...]
    a2 = a_ref[:, D:]
    v = jnp.dot(a2, w_ref[...])
    s = jnp.sum(h * v, axis=1, keepdims=True)
    m = jnp.max(s)
    wexp = jnp.exp(s - m)
    hw_ref[...] = wexp * h
    nrm = jnp.sqrt(jnp.sum(h * h, axis=1, keepdims=True))
    item_ref[...] = h / jnp.maximum(nrm, 1e-12)


def _finish_body(acc2_ref, out_ref):
    acc = acc2_ref[0, :N_PREFIX, :]
    nrm = jnp.sqrt(jnp.sum(acc * acc, axis=1, keepdims=True))
    out_ref[...] = acc / jnp.maximum(nrm, 1e-12)


NBUF = 2
B = 8
C0 = 160
C1 = 0
TOTCH = NS * (C0 + C1)


def _make_scatter():
    def _scatter_body(evc_hbm, euc_hbm, hw_hbm, zeros_hbm, out_hbm,
                      ubv0, ubv1, ubu0, ubu1, rows0, rows1, acc_sh,
                      g0, g1, ri0, ri1):
        ubv = (ubv0, ubv1)
        ubu = (ubu0, ubu1)
        rows = (rows0, rows1)
        g = (g0, g1)
        ri = (ri0, ri1)
        cid = lax.axis_index("c")
        sid = lax.axis_index("s")
        r0 = sid * ROWS_PER_TILE
        c_tile = jnp.where(cid == 0, C0, C1)
        start = sid * C0
        nblk = c_tile // B

        @pl.when(cid == 0)
        def _():
            zc = pltpu.async_copy(zeros_hbm,
                                  acc_sh.at[pl.ds(r0, ROWS_PER_TILE)], g0)
            for t in range(2):
                pltpu.async_copy(evc_hbm.at[pl.ds(start + t * B, B)], ubv[t],
                                 ri[t])
                pltpu.async_copy(euc_hbm.at[pl.ds(start + t * B, B)], ubu[t],
                                 ri[t])
            zc.wait()

        plsc.subcore_barrier()

        @pl.when(cid == 0)
        def _():
            pltpu.make_async_copy(evc_hbm.at[pl.ds(start, B)], ubv0,
                                  ri0).wait()
            pltpu.make_async_copy(euc_hbm.at[pl.ds(start, B)], ubu0,
                                  ri0).wait()
            for t in range(2):
                pltpu.async_copy(hw_hbm.at[ubv0.at[t]], rows[t], g[t])

        @pl.loop(0, nblk, step=2)
        def _blocks(j):
            for blk in range(2):
                base = (j + blk) * B
                for k in range(B):
                    b = k % NBUF
                    cur = base + k
                    pltpu.make_async_copy(hw_hbm.at[ubv[blk].at[k]], rows[b],
                                          g[b]).wait()
                    pltpu.sync_copy(rows[b], acc_sh.at[ubu[blk].at[k]],
                                    add=True)
                    if k == B - 2:
                        @pl.when(base + B < c_tile)
                        def _():
                            pltpu.make_async_copy(
                                evc_hbm.at[pl.ds(start, B)], ubv[1 - blk],
                                ri[1 - blk]).wait()
                            pltpu.make_async_copy(
                                euc_hbm.at[pl.ds(start, B)], ubu[1 - blk],
                                ri[1 - blk]).wait()
                    if k < B - 2:
                        @pl.when(cur + NBUF < c_tile)
                        def _():
                            pltpu.async_copy(hw_hbm.at[ubv[blk].at[k + 2]],
                                             rows[b], g[b])
                    else:
                        @pl.when(cur + NBUF < c_tile)
                        def _():
                            pltpu.async_copy(
                                hw_hbm.at[ubv[1 - blk].at[k + 2 - B]],
                                rows[b], g[b])
                @pl.when(base + 2 * B < c_tile)
                def _():
                    off = start + base + 2 * B
                    pltpu.async_copy(evc_hbm.at[pl.ds(off, B)], ubv[blk],
                                     ri[blk])
                    pltpu.async_copy(euc_hbm.at[pl.ds(off, B)], ubu[blk],
                                     ri[blk])

        plsc.subcore_barrier()

        @pl.when(cid == 0)
        def _():
            pltpu.sync_copy(acc_sh.at[pl.ds(r0, ROWS_PER_TILE)],
                            out_hbm.at[0, pl.ds(r0, ROWS_PER_TILE)])

    return pl.kernel(
        _scatter_body,
        out_type=jax.ShapeDtypeStruct((1, NPAD, D), jnp.float32),
        mesh=plsc.VectorSubcoreMesh(core_axis_name="c", subcore_axis_name="s"),
        scratch_types=[
            pltpu.VMEM((B, CHUNK), jnp.int32),
            pltpu.VMEM((B, CHUNK), jnp.int32),
            pltpu.VMEM((B, CHUNK), jnp.int32),
            pltpu.VMEM((B, CHUNK), jnp.int32),
            pltpu.VMEM((CHUNK, D), jnp.float32),
            pltpu.VMEM((CHUNK, D), jnp.float32),
            pltpu.VMEM_SHARED((NPAD, D), jnp.float32),
            pltpu.SemaphoreType.DMA,
            pltpu.SemaphoreType.DMA,
            pltpu.SemaphoreType.DMA,
            pltpu.SemaphoreType.DMA,
        ],
    )


def kernel(h_prefix, h_item, edge_u, edge_v, W_w, W_b, a_w, a_b):
    eu = edge_u.astype(jnp.int32)
    ev = edge_v.astype(jnp.int32)
    e = eu.shape[0]
    epad = TOTCH * CHUNK
    pad = epad - e
    eu_p = jnp.concatenate([eu, jnp.full((pad,), DUMMY_ROW, jnp.int32)])
    ev_p = jnp.concatenate([ev, jnp.zeros((pad,), jnp.int32)])
    evc = ev_p.reshape(TOTCH, CHUNK)
    euc = eu_p.reshape(TOTCH, CHUNK)
    zeros = jnp.zeros((ROWS_PER_TILE, D), jnp.float32)

    hw, item_out = pl.pallas_call(
        _prep_body,
        out_shape=[
            jax.ShapeDtypeStruct((N_ITEM, D), jnp.float32),
            jax.ShapeDtypeStruct((N_ITEM, D), jnp.float32),
        ],
    )(h_item, W_w, a_w)

    acc2 = _make_scatter()(evc, euc, hw, zeros)

    prefix_out = pl.pallas_call(
        _finish_body,
        out_shape=jax.ShapeDtypeStruct((N_PREFIX, D), jnp.float32),
    )(acc2)
    return prefix_out, item_out


# speedup vs baseline: 1.2173x; 1.2173x over previous
import jax
import jax.numpy as jnp
from jax import lax
from jax.experimental import pallas as pl
from jax.experimental.pallas import tpu as pltpu
from jax.experimental.pallas import tpu_sc as plsc

N_PREFIX = 10000
N_ITEM = 10000
D = 128
NC, NS = 2, 16
NW = NC * NS
CHUNK = 128
ROWS_PER_TILE = 632
NPAD = NS * ROWS_PER_TILE
DUMMY_ROW = NPAD - 1


def _prep_body(h_ref, w_ref, a_ref, hw_ref, item_ref):
    h = h_ref[...]
    a2 = a_ref[:, D:]
    v = jnp.dot(a2, w_ref[...])
    s = jnp.sum(h * v, axis=1, keepdims=True)
    m = jnp.max(s)
    wexp = jnp.exp(s - m)
    hw_ref[...] = wexp * h
    nrm = jnp.sqrt(jnp.sum(h * h, axis=1, keepdims=True))
    item_ref[...] = h / jnp.maximum(nrm, 1e-12)


def _finish_body(acc2_ref, out_ref):
    acc = acc2_ref[0, :N_PREFIX, :] + acc2_ref[1, :N_PREFIX, :]
    nrm = jnp.sqrt(jnp.sum(acc * acc, axis=1, keepdims=True))
    out_ref[...] = acc / jnp.maximum(nrm, 1e-12)


NBUF = 2
B = 8
C0 = 112
C1 = 48
TOTCH = NS * (C0 + C1)


def _make_scatter():
    def _scatter_body(evc_hbm, euc_hbm, hw_hbm, zeros_hbm, out_hbm,
                      ubv0, ubv1, ubu0, ubu1, rows0, rows1, acc_sh,
                      g0, g1, ri0, ri1):
        ubv = (ubv0, ubv1)
        ubu = (ubu0, ubu1)
        rows = (rows0, rows1)
        g = (g0, g1)
        ri = (ri0, ri1)
        cid = lax.axis_index("c")
        sid = lax.axis_index("s")
        r0 = sid * ROWS_PER_TILE
        c_tile = jnp.where(cid == 0, C0, C1)
        start = jnp.where(cid == 0, sid * C0, NS * C0 + sid * C1)
        nblk = c_tile // B

        zc = pltpu.async_copy(zeros_hbm, acc_sh.at[pl.ds(r0, ROWS_PER_TILE)],
                              g0)
        for t in range(2):
            pltpu.async_copy(evc_hbm.at[pl.ds(start + t * B, B)], ubv[t],
                             ri[t])
            pltpu.async_copy(euc_hbm.at[pl.ds(start + t * B, B)], ubu[t],
                             ri[t])
        zc.wait()
        plsc.subcore_barrier()

        pltpu.make_async_copy(evc_hbm.at[pl.ds(start, B)], ubv0, ri0).wait()
        pltpu.make_async_copy(euc_hbm.at[pl.ds(start, B)], ubu0, ri0).wait()
        for t in range(2):
            pltpu.async_copy(hw_hbm.at[ubv0.at[t]], rows[t], g[t])

        @pl.loop(0, nblk, step=2)
        def _blocks(j):
            for blk in range(2):
                base = (j + blk) * B
                for k in range(B):
                    b = k % NBUF
                    cur = base + k
                    pltpu.make_async_copy(hw_hbm.at[ubv[blk].at[k]], rows[b],
                                          g[b]).wait()
                    pltpu.sync_copy(rows[b], acc_sh.at[ubu[blk].at[k]],
                                    add=True)
                    if k == B - 2:
                        @pl.when(base + B < c_tile)
                        def _():
                            pltpu.make_async_copy(
                                evc_hbm.at[pl.ds(start, B)], ubv[1 - blk],
                                ri[1 - blk]).wait()
                            pltpu.make_async_copy(
                                euc_hbm.at[pl.ds(start, B)], ubu[1 - blk],
                                ri[1 - blk]).wait()
                    if k < B - 2:
                        @pl.when(cur + NBUF < c_tile)
                        def _():
                            pltpu.async_copy(hw_hbm.at[ubv[blk].at[k + 2]],
                                             rows[b], g[b])
                    else:
                        @pl.when(cur + NBUF < c_tile)
                        def _():
                            pltpu.async_copy(
                                hw_hbm.at[ubv[1 - blk].at[k + 2 - B]],
                                rows[b], g[b])
                @pl.when(base + 2 * B < c_tile)
                def _():
                    off = start + base + 2 * B
                    pltpu.async_copy(evc_hbm.at[pl.ds(off, B)], ubv[blk],
                                     ri[blk])
                    pltpu.async_copy(euc_hbm.at[pl.ds(off, B)], ubu[blk],
                                     ri[blk])

        plsc.subcore_barrier()
        pltpu.sync_copy(acc_sh.at[pl.ds(r0, ROWS_PER_TILE)],
                        out_hbm.at[cid, pl.ds(r0, ROWS_PER_TILE)])

    return pl.kernel(
        _scatter_body,
        out_type=jax.ShapeDtypeStruct((NC, NPAD, D), jnp.float32),
        mesh=plsc.VectorSubcoreMesh(core_axis_name="c", subcore_axis_name="s"),
        scratch_types=[
            pltpu.VMEM((B, CHUNK), jnp.int32),
            pltpu.VMEM((B, CHUNK), jnp.int32),
            pltpu.VMEM((B, CHUNK), jnp.int32),
            pltpu.VMEM((B, CHUNK), jnp.int32),
            pltpu.VMEM((CHUNK, D), jnp.float32),
            pltpu.VMEM((CHUNK, D), jnp.float32),
            pltpu.VMEM_SHARED((NPAD, D), jnp.float32),
            pltpu.SemaphoreType.DMA,
            pltpu.SemaphoreType.DMA,
            pltpu.SemaphoreType.DMA,
            pltpu.SemaphoreType.DMA,
        ],
    )


def kernel(h_prefix, h_item, edge_u, edge_v, W_w, W_b, a_w, a_b):
    eu = edge_u.astype(jnp.int32)
    ev = edge_v.astype(jnp.int32)
    e = eu.shape[0]
    epad = TOTCH * CHUNK
    pad = epad - e
    spare = NPAD - N_PREFIX
    dummy_u = N_PREFIX + jnp.arange(pad, dtype=jnp.int32) % spare
    eu_p = jnp.concatenate([eu, dummy_u])
    ev_p = jnp.concatenate([ev, jnp.zeros((pad,), jnp.int32)])
    evc = ev_p.reshape(TOTCH, CHUNK)
    euc = eu_p.reshape(TOTCH, CHUNK)
    zeros = jnp.zeros((ROWS_PER_TILE, D), jnp.float32)

    hw, item_out = pl.pallas_call(
        _prep_body,
        out_shape=[
            jax.ShapeDtypeStruct((N_ITEM, D), jnp.float32),
            jax.ShapeDtypeStruct((N_ITEM, D), jnp.float32),
        ],
    )(h_item, W_w, a_w)

    acc2 = _make_scatter()(evc, euc, hw, zeros)

    prefix_out = pl.pallas_call(
        _finish_body,
        out_shape=jax.ShapeDtypeStruct((N_PREFIX, D), jnp.float32),
    )(acc2)
    return prefix_out, item_out


# speedup vs baseline: 1.2177x; 1.0003x over previous
import jax
import jax.numpy as jnp
from jax import lax
from jax.experimental import pallas as pl
from jax.experimental.pallas import tpu as pltpu
from jax.experimental.pallas import tpu_sc as plsc

N_PREFIX = 10000
N_ITEM = 10000
D = 128
NC, NS = 2, 16
NW = NC * NS
CHUNK = 128
ROWS_PER_TILE = 632
NPAD = NS * ROWS_PER_TILE
DUMMY_ROW = NPAD - 1


def _prep_body(h_ref, w_ref, a_ref, hw_ref, item_ref):
    h = h_ref[...]
    a2 = a_ref[:, D:]
    v = jnp.dot(a2, w_ref[...])
    s = jnp.sum(h * v, axis=1, keepdims=True)
    m = jnp.max(s)
    wexp = jnp.exp(s - m)
    hw_ref[...] = wexp * h
    nrm = jnp.sqrt(jnp.sum(h * h, axis=1, keepdims=True))
    item_ref[...] = h / jnp.maximum(nrm, 1e-12)


def _finish_body(acc2_ref, out_ref):
    acc = acc2_ref[0, :N_PREFIX, :] + acc2_ref[1, :N_PREFIX, :]
    nrm = jnp.sqrt(jnp.sum(acc * acc, axis=1, keepdims=True))
    out_ref[...] = acc / jnp.maximum(nrm, 1e-12)


NBUF = 2
B = 8
C0 = 112
C1 = 48
TOTCH = NS * (C0 + C1)


def _make_scatter():
    def _scatter_body(evc_hbm, euc_hbm, hw_hbm, zeros_hbm, out_hbm,
                      ubv0, ubv1, ubu0, ubu1, rows0, rows1, acc_sh,
                      g0, g1, ri0, ri1):
        ubv = (ubv0, ubv1)
        ubu = (ubu0, ubu1)
        rows = (rows0, rows1)
        g = (g0, g1)
        ri = (ri0, ri1)
        cid = lax.axis_index("c")
        sid = lax.axis_index("s")
        r0 = sid * ROWS_PER_TILE
        c_tile = jnp.where(cid == 0, C0, C1)
        start = jnp.where(cid == 0, sid * C0, NS * C0 + sid * C1)
        nblk = c_tile // B

        zc = pltpu.async_copy(zeros_hbm, acc_sh.at[pl.ds(r0, ROWS_PER_TILE)],
                              g0)
        for t in range(2):
            pltpu.async_copy(evc_hbm.at[pl.ds(start + t * B, B)], ubv[t],
                             ri[t])
            pltpu.async_copy(euc_hbm.at[pl.ds(start + t * B, B)], ubu[t],
                             ri[t])
        with jax.named_scope("sc_zero"):
            zc.wait()
        with jax.named_scope("sc_bar1"):
            plsc.subcore_barrier()

        pltpu.make_async_copy(evc_hbm.at[pl.ds(start, B)], ubv0, ri0).wait()
        pltpu.make_async_copy(euc_hbm.at[pl.ds(start, B)], ubu0, ri0).wait()
        for t in range(2):
            pltpu.async_copy(hw_hbm.at[ubv0.at[t]], rows[t], g[t])

        @pl.loop(0, nblk, step=2)
        def _blocks(j):
          with jax.named_scope("sc_loop"):
            for blk in range(2):
                base = (j + blk) * B
                for k in range(B):
                    b = k % NBUF
                    cur = base + k
                    pltpu.make_async_copy(hw_hbm.at[ubv[blk].at[k]], rows[b],
                                          g[b]).wait()
                    pltpu.sync_copy(rows[b], acc_sh.at[ubu[blk].at[k]],
                                    add=True)
                    if k == B - 2:
                        @pl.when(base + B < c_tile)
                        def _():
                            pltpu.make_async_copy(
                                evc_hbm.at[pl.ds(start, B)], ubv[1 - blk],
                                ri[1 - blk]).wait()
                            pltpu.make_async_copy(
                                euc_hbm.at[pl.ds(start, B)], ubu[1 - blk],
                                ri[1 - blk]).wait()
                    if k < B - 2:
                        @pl.when(cur + NBUF < c_tile)
                        def _():
                            pltpu.async_copy(hw_hbm.at[ubv[blk].at[k + 2]],
                                             rows[b], g[b])
                    else:
                        @pl.when(cur + NBUF < c_tile)
                        def _():
                            pltpu.async_copy(
                                hw_hbm.at[ubv[1 - blk].at[k + 2 - B]],
                                rows[b], g[b])
                @pl.when(base + 2 * B < c_tile)
                def _():
                    off = start + base + 2 * B
                    pltpu.async_copy(evc_hbm.at[pl.ds(off, B)], ubv[blk],
                                     ri[blk])
                    pltpu.async_copy(euc_hbm.at[pl.ds(off, B)], ubu[blk],
                                     ri[blk])

        with jax.named_scope("sc_bar2"):
            plsc.subcore_barrier()
        with jax.named_scope("sc_wb"):
            pltpu.sync_copy(acc_sh.at[pl.ds(r0, ROWS_PER_TILE)],
                            out_hbm.at[cid, pl.ds(r0, ROWS_PER_TILE)])

    return pl.kernel(
        _scatter_body,
        out_type=jax.ShapeDtypeStruct((NC, NPAD, D), jnp.float32),
        mesh=plsc.VectorSubcoreMesh(core_axis_name="c", subcore_axis_name="s"),
        scratch_types=[
            pltpu.VMEM((B, CHUNK), jnp.int32),
            pltpu.VMEM((B, CHUNK), jnp.int32),
            pltpu.VMEM((B, CHUNK), jnp.int32),
            pltpu.VMEM((B, CHUNK), jnp.int32),
            pltpu.VMEM((CHUNK, D), jnp.float32),
            pltpu.VMEM((CHUNK, D), jnp.float32),
            pltpu.VMEM_SHARED((NPAD, D), jnp.float32),
            pltpu.SemaphoreType.DMA,
            pltpu.SemaphoreType.DMA,
            pltpu.SemaphoreType.DMA,
            pltpu.SemaphoreType.DMA,
        ],
    )


def kernel(h_prefix, h_item, edge_u, edge_v, W_w, W_b, a_w, a_b):
    eu = edge_u.astype(jnp.int32)
    ev = edge_v.astype(jnp.int32)
    e = eu.shape[0]
    epad = TOTCH * CHUNK
    pad = epad - e
    spare = NPAD - N_PREFIX
    dummy_u = N_PREFIX + jnp.arange(pad, dtype=jnp.int32) % spare
    eu_p = jnp.concatenate([eu, dummy_u])
    ev_p = jnp.concatenate([ev, jnp.zeros((pad,), jnp.int32)])
    evc = ev_p.reshape(TOTCH, CHUNK)
    euc = eu_p.reshape(TOTCH, CHUNK)
    zeros = jnp.zeros((ROWS_PER_TILE, D), jnp.float32)

    hw, item_out = pl.pallas_call(
        _prep_body,
        out_shape=[
            jax.ShapeDtypeStruct((N_ITEM, D), jnp.float32),
            jax.ShapeDtypeStruct((N_ITEM, D), jnp.float32),
        ],
    )(h_item, W_w, a_w)

    acc2 = _make_scatter()(evc, euc, hw, zeros)

    prefix_out = pl.pallas_call(
        _finish_body,
        out_shape=jax.ShapeDtypeStruct((N_PREFIX, D), jnp.float32),
    )(acc2)
    return prefix_out, item_out


# speedup vs baseline: 4.0112x; 3.2941x over previous
import jax
import jax.numpy as jnp
from jax import lax
from jax.experimental import pallas as pl
from jax.experimental.pallas import tpu as pltpu
from jax.experimental.pallas import tpu_sc as plsc

N_PREFIX = 10000
N_ITEM = 10000
D = 128
NC, NS = 2, 16
NW = NC * NS
CHUNK = 128
ROWS_PER_TILE = 632
NPAD = NS * ROWS_PER_TILE
DUMMY_ROW = NPAD - 1


def _prep_body(h_ref, w_ref, a_ref, hw_ref, item_ref):
    h = h_ref[...]
    a2 = a_ref[:, D:]
    v = jnp.dot(a2, w_ref[...])
    s = jnp.sum(h * v, axis=1, keepdims=True)
    m = jnp.max(s)
    wexp = jnp.exp(s - m)
    hw_ref[...] = wexp * h
    nrm = jnp.sqrt(jnp.sum(h * h, axis=1, keepdims=True))
    item_ref[...] = h / jnp.maximum(nrm, 1e-12)


def _finish_body(acc2_ref, out_ref):
    acc = acc2_ref[0, :N_PREFIX, :] + acc2_ref[1, :N_PREFIX, :]
    nrm = jnp.sqrt(jnp.sum(acc * acc, axis=1, keepdims=True))
    out_ref[...] = acc / jnp.maximum(nrm, 1e-12)


NBUF = 2
B = 8
C0 = 80
C1 = 80
TOTCH = NS * (C0 + C1)


def _make_scatter():
    def _scatter_body(evc_hbm, euc_hbm, hw_hbm, zeros_hbm, out_hbm,
                      ubv0, ubv1, ubu0, ubu1, rows0, rows1, acc_sh,
                      g0, g1, ri0, ri1):
        ubv = (ubv0, ubv1)
        ubu = (ubu0, ubu1)
        rows = (rows0, rows1)
        g = (g0, g1)
        ri = (ri0, ri1)
        cid = lax.axis_index("c")
        sid = lax.axis_index("s")
        r0 = sid * ROWS_PER_TILE
        c_tile = jnp.where(cid == 0, C0, C1)
        start = jnp.where(cid == 0, sid * C0, NS * C0 + sid * C1)
        nblk = c_tile // B

        zc = pltpu.async_copy(zeros_hbm, acc_sh.at[pl.ds(r0, ROWS_PER_TILE)],
                              g0)
        for t in range(2):
            pltpu.async_copy(evc_hbm.at[pl.ds(start + t * B, B)], ubv[t],
                             ri[t])
            pltpu.async_copy(euc_hbm.at[pl.ds(start + t * B, B)], ubu[t],
                             ri[t])
        zc.wait()
        plsc.subcore_barrier()

        pltpu.make_async_copy(evc_hbm.at[pl.ds(start, B)], ubv0, ri0).wait()
        pltpu.make_async_copy(euc_hbm.at[pl.ds(start, B)], ubu0, ri0).wait()
        for t in range(2):
            pltpu.async_copy(hw_hbm.at[ubv0.at[t]], rows[t], g[t])

        @pl.loop(0, nblk, step=2)
        def _blocks(j):
            for blk in range(2):
                base = (j + blk) * B
                for k in range(B):
                    b = k % NBUF
                    cur = base + k
                    pltpu.make_async_copy(hw_hbm.at[ubv[blk].at[k]], rows[b],
                                          g[b]).wait()
                    pltpu.sync_copy(rows[b], acc_sh.at[ubu[blk].at[k]],
                                    add=True)
                    if k == B - 2:
                        @pl.when(base + B < c_tile)
                        def _():
                            pltpu.make_async_copy(
                                evc_hbm.at[pl.ds(start, B)], ubv[1 - blk],
                                ri[1 - blk]).wait()
                            pltpu.make_async_copy(
                                euc_hbm.at[pl.ds(start, B)], ubu[1 - blk],
                                ri[1 - blk]).wait()
                    if k < B - 2:
                        @pl.when(cur + NBUF < c_tile)
                        def _():
                            pltpu.async_copy(hw_hbm.at[ubv[blk].at[k + 2]],
                                             rows[b], g[b])
                    else:
                        @pl.when(cur + NBUF < c_tile)
                        def _():
                            pltpu.async_copy(
                                hw_hbm.at[ubv[1 - blk].at[k + 2 - B]],
                                rows[b], g[b])
                @pl.when(base + 2 * B < c_tile)
                def _():
                    off = start + base + 2 * B
                    pltpu.async_copy(evc_hbm.at[pl.ds(off, B)], ubv[blk],
                                     ri[blk])
                    pltpu.async_copy(euc_hbm.at[pl.ds(off, B)], ubu[blk],
                                     ri[blk])

        plsc.subcore_barrier()
        pltpu.sync_copy(acc_sh.at[pl.ds(r0, ROWS_PER_TILE)],
                        out_hbm.at[cid, pl.ds(r0, ROWS_PER_TILE)])

    return pl.kernel(
        _scatter_body,
        out_type=jax.ShapeDtypeStruct((NC, NPAD, D), jnp.float32),
        mesh=plsc.VectorSubcoreMesh(core_axis_name="c", subcore_axis_name="s"),
        scratch_types=[
            pltpu.VMEM((B, CHUNK), jnp.int32),
            pltpu.VMEM((B, CHUNK), jnp.int32),
            pltpu.VMEM((B, CHUNK), jnp.int32),
            pltpu.VMEM((B, CHUNK), jnp.int32),
            pltpu.VMEM((CHUNK, D), jnp.float32),
            pltpu.VMEM((CHUNK, D), jnp.float32),
            pltpu.VMEM_SHARED((NPAD, D), jnp.float32),
            pltpu.SemaphoreType.DMA,
            pltpu.SemaphoreType.DMA,
            pltpu.SemaphoreType.DMA,
            pltpu.SemaphoreType.DMA,
        ],
    )


def kernel(h_prefix, h_item, edge_u, edge_v, W_w, W_b, a_w, a_b):
    eu = edge_u.astype(jnp.int32)
    ev = edge_v.astype(jnp.int32)
    e = eu.shape[0]
    epad = TOTCH * CHUNK
    pad = epad - e
    spare = NPAD - N_PREFIX
    dummy_u = N_PREFIX + jnp.arange(pad, dtype=jnp.int32) % spare
    eu_p = jnp.concatenate([eu, dummy_u])
    dummy_v = jnp.arange(pad, dtype=jnp.int32) % N_ITEM
    ev_p = jnp.concatenate([ev, dummy_v])
    evc = ev_p.reshape(TOTCH, CHUNK)
    euc = eu_p.reshape(TOTCH, CHUNK)
    zeros = jnp.zeros((ROWS_PER_TILE, D), jnp.float32)

    hw, item_out = pl.pallas_call(
        _prep_body,
        out_shape=[
            jax.ShapeDtypeStruct((N_ITEM, D), jnp.float32),
            jax.ShapeDtypeStruct((N_ITEM, D), jnp.float32),
        ],
    )(h_item, W_w, a_w)

    acc2 = _make_scatter()(evc, euc, hw, zeros)

    prefix_out = pl.pallas_call(
        _finish_body,
        out_shape=jax.ShapeDtypeStruct((N_PREFIX, D), jnp.float32),
    )(acc2)
    return prefix_out, item_out


# speedup vs baseline: 4.0493x; 1.0095x over previous
import jax
import jax.numpy as jnp
from jax import lax
from jax.experimental import pallas as pl
from jax.experimental.pallas import tpu as pltpu
from jax.experimental.pallas import tpu_sc as plsc

N_PREFIX = 10000
N_ITEM = 10000
D = 128
NC, NS = 2, 16
NW = NC * NS
CHUNK = 128
ROWS_PER_TILE = 632
NPAD = NS * ROWS_PER_TILE
DUMMY_ROW = NPAD - 1


def _prep_body(h_ref, w_ref, a_ref, hw_ref, item_ref):
    h = h_ref[...]
    a2 = a_ref[:, D:]
    v = jnp.dot(a2, w_ref[...])
    s = jnp.sum(h * v, axis=1, keepdims=True)
    m = jnp.max(s)
    wexp = jnp.exp(s - m)
    hw_ref[...] = wexp * h
    nrm = jnp.sqrt(jnp.sum(h * h, axis=1, keepdims=True))
    item_ref[...] = h / jnp.maximum(nrm, 1e-12)


def _finish_body(acc2_ref, out_ref):
    acc = acc2_ref[0, :N_PREFIX, :] + acc2_ref[1, :N_PREFIX, :]
    nrm = jnp.sqrt(jnp.sum(acc * acc, axis=1, keepdims=True))
    out_ref[...] = acc / jnp.maximum(nrm, 1e-12)


NBUF = 2
B = 8
C0 = 80
C1 = 80
TOTCH = NS * (C0 + C1)


def _make_scatter():
    def _scatter_body(evc_hbm, euc_hbm, hw_hbm, zeros_hbm, out_hbm,
                      ubv0, ubv1, ubu0, ubu1, rows0, rows1, acc_sh,
                      g0, g1, ri0, ri1, zsem):
        ubv = (ubv0, ubv1)
        ubu = (ubu0, ubu1)
        rows = (rows0, rows1)
        g = (g0, g1)
        ri = (ri0, ri1)
        cid = lax.axis_index("c")
        sid = lax.axis_index("s")
        r0 = sid * ROWS_PER_TILE
        c_tile = jnp.where(cid == 0, C0, C1)
        start = jnp.where(cid == 0, sid * C0, NS * C0 + sid * C1)
        nblk = c_tile // B

        zc = pltpu.async_copy(zeros_hbm, acc_sh.at[pl.ds(r0, ROWS_PER_TILE)],
                              zsem)
        for t in range(2):
            pltpu.async_copy(evc_hbm.at[pl.ds(start + t * B, B)], ubv[t],
                             ri[t])
            pltpu.async_copy(euc_hbm.at[pl.ds(start + t * B, B)], ubu[t],
                             ri[t])
        pltpu.make_async_copy(evc_hbm.at[pl.ds(start, B)], ubv0, ri0).wait()
        pltpu.make_async_copy(euc_hbm.at[pl.ds(start, B)], ubu0, ri0).wait()
        for t in range(2):
            pltpu.async_copy(hw_hbm.at[ubv0.at[t]], rows[t], g[t])
        zc.wait()
        plsc.subcore_barrier()

        @pl.loop(0, nblk, step=2)
        def _blocks(j):
            for blk in range(2):
                base = (j + blk) * B
                for k in range(B):
                    b = k % NBUF
                    cur = base + k
                    pltpu.make_async_copy(hw_hbm.at[ubv[blk].at[k]], rows[b],
                                          g[b]).wait()
                    pltpu.sync_copy(rows[b], acc_sh.at[ubu[blk].at[k]],
                                    add=True)
                    if k == B - 2:
                        @pl.when(base + B < c_tile)
                        def _():
                            pltpu.make_async_copy(
                                evc_hbm.at[pl.ds(start, B)], ubv[1 - blk],
                                ri[1 - blk]).wait()
                            pltpu.make_async_copy(
                                euc_hbm.at[pl.ds(start, B)], ubu[1 - blk],
                                ri[1 - blk]).wait()
                    if k < B - 2:
                        @pl.when(cur + NBUF < c_tile)
                        def _():
                            pltpu.async_copy(hw_hbm.at[ubv[blk].at[k + 2]],
                                             rows[b], g[b])
                    else:
                        @pl.when(cur + NBUF < c_tile)
                        def _():
                            pltpu.async_copy(
                                hw_hbm.at[ubv[1 - blk].at[k + 2 - B]],
                                rows[b], g[b])
                @pl.when(base + 2 * B < c_tile)
                def _():
                    off = start + base + 2 * B
                    pltpu.async_copy(evc_hbm.at[pl.ds(off, B)], ubv[blk],
                                     ri[blk])
                    pltpu.async_copy(euc_hbm.at[pl.ds(off, B)], ubu[blk],
                                     ri[blk])

        plsc.subcore_barrier()
        pltpu.sync_copy(acc_sh.at[pl.ds(r0, ROWS_PER_TILE)],
                        out_hbm.at[cid, pl.ds(r0, ROWS_PER_TILE)])

    return pl.kernel(
        _scatter_body,
        out_type=jax.ShapeDtypeStruct((NC, NPAD, D), jnp.float32),
        mesh=plsc.VectorSubcoreMesh(core_axis_name="c", subcore_axis_name="s"),
        scratch_types=[
            pltpu.VMEM((B, CHUNK), jnp.int32),
            pltpu.VMEM((B, CHUNK), jnp.int32),
            pltpu.VMEM((B, CHUNK), jnp.int32),
            pltpu.VMEM((B, CHUNK), jnp.int32),
            pltpu.VMEM((CHUNK, D), jnp.float32),
            pltpu.VMEM((CHUNK, D), jnp.float32),
            pltpu.VMEM_SHARED((NPAD, D), jnp.float32),
            pltpu.SemaphoreType.DMA,
            pltpu.SemaphoreType.DMA,
            pltpu.SemaphoreType.DMA,
            pltpu.SemaphoreType.DMA,
            pltpu.SemaphoreType.DMA,
        ],
    )


def kernel(h_prefix, h_item, edge_u, edge_v, W_w, W_b, a_w, a_b):
    eu = edge_u.astype(jnp.int32)
    ev = edge_v.astype(jnp.int32)
    e = eu.shape[0]
    epad = TOTCH * CHUNK
    pad = epad - e
    spare = NPAD - N_PREFIX
    dummy_u = N_PREFIX + jnp.arange(pad, dtype=jnp.int32) % spare
    eu_p = jnp.concatenate([eu, dummy_u])
    dummy_v = jnp.arange(pad, dtype=jnp.int32) % N_ITEM
    ev_p = jnp.concatenate([ev, dummy_v])
    evc = ev_p.reshape(TOTCH, CHUNK)
    euc = eu_p.reshape(TOTCH, CHUNK)
    zeros = jnp.zeros((ROWS_PER_TILE, D), jnp.float32)

    hw, item_out = pl.pallas_call(
        _prep_body,
        out_shape=[
            jax.ShapeDtypeStruct((N_ITEM, D), jnp.float32),
            jax.ShapeDtypeStruct((N_ITEM, D), jnp.float32),
        ],
    )(h_item, W_w, a_w)

    acc2 = _make_scatter()(evc, euc, hw, zeros)

    prefix_out = pl.pallas_call(
        _finish_body,
        out_shape=jax.ShapeDtypeStruct((N_PREFIX, D), jnp.float32),
    )(acc2)
    return prefix_out, item_out
